# baseline (speedup 1.0000x reference)
import functools

import jax
import jax.numpy as jnp
from jax.experimental import pallas as pl
from jax.experimental.pallas import tpu as pltpu

H = 128
_BM = 256


def _act(x, kind):
    if kind == "none":
        return x
    if kind == "silu":
        return x * jax.nn.sigmoid(x)
    if kind == "relu":
        return jnp.maximum(x, 0.0)
    if kind == "leaky01":
        return jnp.where(x > 0, x, 0.01 * x)
    if kind == "leaky2":
        return jnp.where(x > 0, x, 0.2 * x)
    raise ValueError(kind)


def _dense_body(act, nx, has_b2, has_add, *refs):
    acc = jnp.dot(refs[0][...], refs[1][...], preferred_element_type=jnp.float32, precision=jax.lax.Precision.HIGHEST)
    i = 2
    if nx == 2:
        acc = acc + jnp.dot(refs[2][...], refs[3][...],
                            preferred_element_type=jnp.float32, precision=jax.lax.Precision.HIGHEST)
        i = 4
    acc = acc + refs[i][...]
    i += 1
    if has_b2:
        acc = acc + refs[i][...] * refs[i + 1][...]
        i += 2
    if has_add:
        acc = acc + refs[i][...]
        i += 1
    refs[i][...] = _act(acc, act)


def _pad_rows(a, m):
    r = (-a.shape[0]) % m
    if r:
        a = jnp.pad(a, ((0, r),) + ((0, 0),) * (a.ndim - 1))
    return a


def _pad_to(a, axis, mult):
    r = (-a.shape[axis]) % mult
    if r:
        pw = [(0, 0)] * a.ndim
        pw[axis] = (0, r)
        a = jnp.pad(a, pw)
    return a


def _dense(x1, w1, b=None, act="none", x2=None, w2=None, mvec=None, b2=None,
           add=None):
    m, nout = x1.shape[0], w1.shape[1]
    x1 = _pad_to(_pad_rows(x1, _BM), 1, 128)
    w1 = _pad_to(_pad_to(w1, 0, 128), 1, 128)
    mp, npad = x1.shape[0], w1.shape[1]
    nx = 1
    ops = [x1, w1]
    specs = [pl.BlockSpec((_BM, x1.shape[1]), lambda i: (i, 0)),
             pl.BlockSpec(w1.shape, lambda i: (0, 0))]
    if x2 is not None:
        x2 = _pad_to(_pad_rows(x2, _BM), 1, 128)
        w2 = _pad_to(_pad_to(w2, 0, 128), 1, 128)
        nx = 2
        ops += [x2, w2]
        specs += [pl.BlockSpec((_BM, x2.shape[1]), lambda i: (i, 0)),
                  pl.BlockSpec(w2.shape, lambda i: (0, 0))]
    if b is None:
        b = jnp.zeros((nout,), jnp.float32)
    b = _pad_to(b.reshape(1, -1), 1, 128)
    ops.append(b)
    specs.append(pl.BlockSpec(b.shape, lambda i: (0, 0)))
    if b2 is not None:
        mv = _pad_rows(mvec.reshape(-1, 1), _BM)
        b2p = _pad_to(b2.reshape(1, -1), 1, 128)
        ops += [mv, b2p]
        specs += [pl.BlockSpec((_BM, 1), lambda i: (i, 0)),
                  pl.BlockSpec(b2p.shape, lambda i: (0, 0))]
    if add is not None:
        addp = _pad_to(_pad_rows(add, _BM), 1, 128)
        ops.append(addp)
        specs.append(pl.BlockSpec((_BM, npad), lambda i: (i, 0)))
    out = pl.pallas_call(
        functools.partial(_dense_body, act, nx, b2 is not None, add is not None),
        grid=(mp // _BM,),
        in_specs=specs,
        out_specs=pl.BlockSpec((_BM, npad), lambda i: (i, 0)),
        out_shape=jax.ShapeDtypeStruct((mp, npad), jnp.float32),
    )(*ops)
    return out[:m, :nout]


def _gru_body(elu_in, relu_out, x_ref, bin_ref, h_ref, wih_ref, whh_ref,
              bih_ref, bhh_ref, out_ref):
    x = x_ref[...] + bin_ref[...]
    if elu_in:
        x = jnp.where(x > 0, x, jnp.exp(jnp.minimum(x, 0.0)) - 1.0)
    h = h_ref[...]
    gi = jnp.dot(x, wih_ref[...], preferred_element_type=jnp.float32, precision=jax.lax.Precision.HIGHEST) + bih_ref[...]
    gh = jnp.dot(h, whh_ref[...], preferred_element_type=jnp.float32, precision=jax.lax.Precision.HIGHEST) + bhh_ref[...]
    r = jax.nn.sigmoid(gi[:, :H] + gh[:, :H])
    zt = jax.nn.sigmoid(gi[:, H:2 * H] + gh[:, H:2 * H])
    n = jnp.tanh(gi[:, 2 * H:] + r * gh[:, 2 * H:])
    o = (1.0 - zt) * n + zt * h
    if relu_out:
        o = jnp.maximum(o, 0.0)
    out_ref[...] = o


def _gru(p, inp, h, in_bias=None, elu_in=True, relu_out=True):
    m = inp.shape[0]
    inp = _pad_rows(inp, _BM)
    hp = _pad_rows(h, _BM)
    mp = inp.shape[0]
    if in_bias is None:
        in_bias = jnp.zeros((H,), jnp.float32)
    ops = [inp, in_bias.reshape(1, H), hp, p["Wih"], p["Whh"],
           p["bih"].reshape(1, -1), p["bhh"].reshape(1, -1)]
    specs = [pl.BlockSpec((_BM, H), lambda i: (i, 0)),
             pl.BlockSpec((1, H), lambda i: (0, 0)),
             pl.BlockSpec((_BM, H), lambda i: (i, 0)),
             pl.BlockSpec((H, 3 * H), lambda i: (0, 0)),
             pl.BlockSpec((H, 3 * H), lambda i: (0, 0)),
             pl.BlockSpec((1, 3 * H), lambda i: (0, 0)),
             pl.BlockSpec((1, 3 * H), lambda i: (0, 0))]
    out = pl.pallas_call(
        functools.partial(_gru_body, elu_in, relu_out),
        grid=(mp // _BM,),
        in_specs=specs,
        out_specs=pl.BlockSpec((_BM, H), lambda i: (i, 0)),
        out_shape=jax.ShapeDtypeStruct((mp, H), jnp.float32),
    )(*ops)
    return out[:m]


def _seg_softmax(score, seg, num):
    m = jax.ops.segment_max(score, seg, num_segments=num)
    m = jnp.where(jnp.isfinite(m), m, 0.0)
    e = jnp.exp(score - m[seg])
    s = jax.ops.segment_sum(e, seg, num_segments=num)
    return e / (s[seg] + 1e-16)


def kernel(x, edge_attr, pos, label, params, edge_index, z, batch):
    n = x.shape[0]
    b = label.shape[0]
    eg = params["egnn"]
    ap = params["afp"]
    order = jnp.argsort(edge_index[1])
    srcs = edge_index[0][order]
    dsts = edge_index[1][order]
    eas = edge_attr[order]
    pos0 = pos[:, 0, :]

    ones_e = jnp.ones((srcs.shape[0],), jnp.float32)
    deg = jax.ops.segment_sum(ones_e, dsts, num_segments=n)

    h = eg["emb"][z]
    d2 = jnp.sum((pos0[srcs] - pos0[dsts]) ** 2, axis=-1)
    for lp in eg["layers"]:
        p_src = _dense(h, lp["W1"][:H], lp["b1"])
        q_dst = _dense(h, lp["W1"][H:2 * H])
        c_row = lp["W1"][2 * H]
        s = jax.nn.silu(p_src[srcs] + q_dst[dsts] + d2[:, None] * c_row)
        agg0 = jax.ops.segment_sum(s, dsts, num_segments=n)
        agg = _dense(agg0, lp["W2"], mvec=deg, b2=lp["b2"])
        t = _dense(h, lp["U1"][:H], lp["ub1"], "silu", x2=agg, w2=lp["U1"][H:])
        h = _dense(t, lp["U2"], lp["ub2"], add=h)
    g = jax.ops.segment_sum(h, batch, num_segments=b)
    x1 = _dense(g, eg["Wout"], eg["bout"])

    xa = _dense(x, ap["lin1_W"], ap["lin1_b"], "leaky01")
    gp = ap["gate"]
    a_src = _dense(xa, gp["W1"][:H], gp["b1"])
    attr_col = _dense(xa, gp["att_r"].reshape(H, 1))[:, 0]
    w1e = gp["W1"][H]
    xj = jax.nn.leaky_relu(a_src[srcs] + eas[:, None] * w1e, 0.2)
    score = jax.nn.leaky_relu(jnp.sum(xj * gp["att_l"], -1) + attr_col[dsts], 0.2)
    alpha = _seg_softmax(score, dsts, n)
    acc = jax.ops.segment_sum(alpha[:, None] * xj, dsts, num_segments=n)
    ne = (deg > 0).astype(jnp.float32)
    hg = _dense(acc, gp["W2"], gp["bias"], mvec=ne, b2=gp["b2"])
    xa = _gru(ap["gru0"], hg, xa)
    for cp, gpp in zip(ap["convs"], ap["grus"]):
        xt = _dense(xa, cp["W"])
        att2 = jnp.stack([cp["att_src"], cp["att_dst"]], axis=1)
        sd = _dense(xt, att2)
        score = jax.nn.leaky_relu(sd[srcs, 0] + sd[dsts, 1], 0.2)
        alpha = _seg_softmax(score, dsts, n)
        hc = jax.ops.segment_sum(alpha[:, None] * xt[srcs], dsts, num_segments=n)
        xa = _gru(gpp, hc, xa, in_bias=cp["bias"])
    out = jnp.maximum(jax.ops.segment_sum(xa, batch, num_segments=b), 0.0)
    mc = ap["mol_conv"]
    att_s = mc["att_src"].reshape(H, 1)
    att_d = mc["att_dst"].reshape(H, 1)
    for _ in range(3):
        xs = _dense(xa, mc["Ws"])
        sdn = _dense(xs, att_s)[:, 0]
        xd = _dense(out, mc["Wd"])
        adb = _dense(xd, att_d)[:, 0]
        score = jax.nn.leaky_relu(sdn + adb[batch], 0.2)
        alpha = _seg_softmax(score, batch, b)
        hm = jax.ops.segment_sum(alpha[:, None] * xs, batch, num_segments=b)
        out = _gru(ap["mol_gru"], hm, out, in_bias=mc["bias"])
    x2 = _dense(out, ap["lin2_W"], ap["lin2_b"])
    return x1 + x2
